# baseline (speedup 1.0000x reference)
import functools

import jax
import jax.numpy as jnp
from jax import lax
from jax.experimental import pallas as pl
from jax.experimental.pallas import tpu as pltpu
from jax.experimental.pallas import tpu_sc as plsc


def kernel(x, table):
    S, D = table.shape
    info = plsc.get_sparse_core_info()
    NC, NS = info.num_cores, info.num_subcores
    NW = NC * NS
    rows_per_w = S // NW

    mesh = plsc.VectorSubcoreMesh(core_axis_name="c", subcore_axis_name="s")

    @functools.partial(
        pl.kernel,
        mesh=mesh,
        out_type=jax.ShapeDtypeStruct((S, D), jnp.float32),
    )
    def copy_k(table_hbm, out_hbm):
        wid = lax.axis_index("s") * NC + lax.axis_index("c")
        base = wid * rows_per_w
        pltpu.sync_copy(table_hbm.at[pl.ds(base, rows_per_w)],
                        out_hbm.at[pl.ds(base, rows_per_w)])

    return copy_k(table)[None]


# speedup vs baseline: 6.5306x; 6.5306x over previous
import functools

import jax
import jax.numpy as jnp
from jax import lax
from jax.experimental import pallas as pl
from jax.experimental.pallas import tpu as pltpu
from jax.experimental.pallas import tpu_sc as plsc


def kernel(x, table):
    S, D = table.shape
    info = plsc.get_sparse_core_info()
    NC, NS = info.num_cores, info.num_subcores
    NW = NC * NS
    rows_per_w = S // NW

    mesh = plsc.VectorSubcoreMesh(core_axis_name="c", subcore_axis_name="s")

    chunk = rows_per_w // 2

    @functools.partial(
        pl.kernel,
        mesh=mesh,
        out_type=jax.ShapeDtypeStruct((S, D), jnp.float32),
        scratch_types=[
            pltpu.VMEM((2, chunk, D), jnp.float32),
            pltpu.SemaphoreType.DMA,
            pltpu.SemaphoreType.DMA,
        ],
    )
    def copy_k(table_hbm, out_hbm, buf, sem_in, sem_out):
        wid = lax.axis_index("s") * NC + lax.axis_index("c")
        base = wid * rows_per_w
        in0 = pltpu.async_copy(table_hbm.at[pl.ds(base, chunk)], buf.at[0], sem_in)
        in1 = pltpu.async_copy(table_hbm.at[pl.ds(base + chunk, chunk)], buf.at[1], sem_in)
        in0.wait()
        out0 = pltpu.async_copy(buf.at[0], out_hbm.at[pl.ds(base, chunk)], sem_out)
        in1.wait()
        out1 = pltpu.async_copy(buf.at[1], out_hbm.at[pl.ds(base + chunk, chunk)], sem_out)
        out0.wait()
        out1.wait()

    return copy_k(table)[None]


# speedup vs baseline: 20.9822x; 3.2129x over previous
import jax
import jax.numpy as jnp
from jax.experimental import pallas as pl


def kernel(x, table):
    S, D = table.shape

    def body(t_ref, o_ref):
        o_ref[...] = t_ref[...]

    out = pl.pallas_call(
        body,
        out_shape=jax.ShapeDtypeStruct((S, D), jnp.float32),
        grid=(8,),
        in_specs=[pl.BlockSpec((S // 8, D), lambda i: (i, 0))],
        out_specs=pl.BlockSpec((S // 8, D), lambda i: (i, 0)),
    )(table)
    return out[None]
